# Initial kernel scaffold; baseline (speedup 1.0000x reference)
import jax
import jax.numpy as jnp
from jax.experimental import pallas as pl


def kernel(ent_emb, rel_emb, edge_index, cross_edge_index, edge_attr, rel_inv_W, rel_inv_b, bn_gamma, bn_beta, rel_lin_W, rel_lin_b, ent_lin_W, ent_lin_b, s_lin_W, s_lin_b, conv_Wg, conv_bg, conv_Wo, conv_Ws, s_conv_Wg, s_conv_bg, s_conv_Wo, s_conv_Ws):
    raise NotImplementedError("write your pallas kernel here")


# speedup vs baseline: 4.1648x; 4.1648x over previous
import functools

import jax
import jax.numpy as jnp
from jax import lax
from jax.experimental import pallas as pl
from jax.experimental.pallas import tpu as pltpu
from jax.experimental.pallas import tpu_sc as plsc

_NC = 2
_NS = 16
_NW = _NC * _NS
_LANES = 16


def _dot_t(x, w):
    return lax.dot_general(x, w, (((1,), (1,)), ((), ())),
                           preferred_element_type=jnp.float32)


def _norm_rows(x):
    return x / (jnp.sqrt(jnp.sum(x * x, axis=1, keepdims=True)) + 1e-12)


def _rel_kernel(rel_ref, riw_ref, rib_ref, rlw_ref, rlb_ref,
                cwg_ref, cbg_ref, swg_ref, sbg_ref,
                g_ref, rel_final_ref, last_rel_ref):
    r0 = rel_ref[...]
    inv = _dot_t(r0, riw_ref[...]) + rib_ref[...]
    rall0 = jnp.concatenate([r0, inv], axis=0)
    rall1 = _dot_t(rall0, rlw_ref[0]) + rlb_ref[0]
    rall2 = _dot_t(rall1, rlw_ref[1]) + rlb_ref[1]
    g_ref[0] = jax.nn.sigmoid(_dot_t(rall1, cwg_ref[0]) + cbg_ref[0])
    g_ref[1] = jax.nn.sigmoid(_dot_t(rall1, swg_ref[0]) + sbg_ref[0])
    g_ref[2] = jax.nn.sigmoid(_dot_t(rall2, cwg_ref[1]) + cbg_ref[1])
    g_ref[3] = jax.nn.sigmoid(_dot_t(rall2, swg_ref[1]) + sbg_ref[1])
    n0 = _norm_rows(rall0)
    n1 = _norm_rows(rall1)
    n2 = _norm_rows(rall2)
    rel_final_ref[...] = _norm_rows(jnp.concatenate([n0, n1, n2], axis=1))
    last_rel_ref[...] = n2


def _bnlin_kernel(x_ref, s_ref, g_ref, b_ref, ew_ref, eb_ref, sw_ref, sb_ref,
                  lx_ref, slx_ref):
    def bn(x):
        m = jnp.mean(x, axis=0)
        v = jnp.mean((x - m) ** 2, axis=0)
        return (x - m) / jnp.sqrt(v + 1e-5) * g_ref[...] + b_ref[...]

    lx_ref[...] = _dot_t(bn(x_ref[...]), ew_ref[...]) + eb_ref[...]
    slx_ref[...] = _dot_t(bn(s_ref[...]), sw_ref[...]) + sb_ref[...]


def _deginv_kernel(dege_ref, degc_ref, dinve_ref, dinvc_ref):
    dinve_ref[...] = 1.0 / jnp.maximum(
        jnp.sum(dege_ref[...], axis=0), 1.0)[:, None]
    dinvc_ref[...] = 1.0 / jnp.maximum(
        jnp.sum(degc_ref[...], axis=0), 1.0)[:, None]


def _comb_kernel(acc1_ref, acc2_ref, dinve_ref, dinvc_ref, lx_ref, slx_ref,
                 wo1_ref, ws1_ref, wo2_ref, ws2_ref,
                 out_ref, s_out_ref, n_out_ref, n_s_ref):
    agg1 = (acc1_ref[0] + acc1_ref[1]) * dinve_ref[...]
    le = _dot_t(agg1, wo1_ref[...]) + _dot_t(lx_ref[...], ws1_ref[...])
    agg2 = (acc2_ref[0] + acc2_ref[1]) * dinvc_ref[...]
    sle = _dot_t(agg2, wo2_ref[...]) + _dot_t(slx_ref[...], ws2_ref[...])
    out_ref[...] = sle
    s_out_ref[...] = le
    n_out_ref[...] = _norm_rows(sle)
    n_s_ref[...] = _norm_rows(le)


def _final_kernel(ent_ref, no0_ref, ns0_ref, no1_ref, ns1_ref,
                  final_ref, last_ref):
    n0 = _norm_rows(ent_ref[...])
    cat = jnp.concatenate(
        [n0, no0_ref[...], ns0_ref[...], no1_ref[...], ns1_ref[...]], axis=1)
    final_ref[...] = _norm_rows(cat)
    last_ref[...] = _norm_rows(
        jnp.concatenate([no1_ref[...], ns1_ref[...]], axis=1))


def _block_div(nchunk, cap=32):
    cb = 1
    for c in range(1, cap + 1):
        if nchunk % c == 0:
            cb = c
    return cb


@functools.lru_cache(maxsize=None)
def _make_conv_sc(n, d, nblk, cb, ch):
    mesh = plsc.VectorSubcoreMesh(core_axis_name="c", subcore_axis_name="s")
    rows_per_tile = n // _NS

    @functools.partial(
        pl.kernel,
        out_type=jax.ShapeDtypeStruct((_NC, n, d), jnp.float32),
        mesh=mesh,
        scratch_types=[
            pltpu.VMEM((cb, ch), jnp.int32),
            pltpu.VMEM((cb, ch), jnp.int32),
            pltpu.VMEM((cb, ch), jnp.int32),
            pltpu.VMEM((ch, d), jnp.float32),
            pltpu.VMEM((ch, d), jnp.float32),
            pltpu.VMEM_SHARED((n, d), jnp.float32),
            pltpu.SemaphoreType.DMA,
            pltpu.SemaphoreType.DMA,
        ],
    )
    def conv(x_hbm, gate_hbm, src_hbm, dst_hbm, attr_hbm, out_hbm,
             src_v, dst_v, attr_v, xr_v, gr_v, acc_sh, sem1, sem2):
        cid = lax.axis_index("c")
        sid = lax.axis_index("s")
        wid = cid * _NS + sid

        zero = jnp.zeros((_LANES,), jnp.float32)

        def zbody(i, _):
            for j in range(d // _LANES):
                xr_v[i, pl.ds(j * _LANES, _LANES)] = zero
            return 0

        lax.fori_loop(0, ch, zbody, 0)
        row0 = sid * rows_per_tile
        r = 0
        while r < rows_per_tile:
            step = min(ch, rows_per_tile - r)
            pltpu.sync_copy(xr_v.at[pl.ds(0, step)],
                            acc_sh.at[pl.ds(row0 + r, step)])
            r += step
        plsc.subcore_barrier()

        def blk_body(b, _):
            pltpu.sync_copy(src_hbm.at[wid, b], src_v)
            pltpu.sync_copy(dst_hbm.at[wid, b], dst_v)
            pltpu.sync_copy(attr_hbm.at[wid, b], attr_v)

            def chunk_body(c, _):
                cp_x = pltpu.async_copy(x_hbm.at[src_v.at[c]], xr_v, sem1)
                cp_g = pltpu.async_copy(gate_hbm.at[attr_v.at[c]], gr_v, sem2)
                cp_x.wait()
                cp_g.wait()

                def ebody(e, _):
                    for j in range(d // _LANES):
                        sl = pl.ds(j * _LANES, _LANES)
                        xr_v[e, sl] = xr_v[e, sl] * gr_v[e, sl]
                    return 0

                lax.fori_loop(0, ch, ebody, 0)
                pltpu.sync_copy(xr_v, acc_sh.at[dst_v.at[c]], add=True)
                return 0

            lax.fori_loop(0, cb, chunk_body, 0)
            return 0

        lax.fori_loop(0, nblk, blk_body, 0)
        plsc.subcore_barrier()

        pltpu.sync_copy(acc_sh.at[pl.ds(row0, rows_per_tile)],
                        out_hbm.at[cid, pl.ds(row0, rows_per_tile)])

    return conv


@functools.lru_cache(maxsize=None)
def _make_deg_sc(n, nchunk, ch):
    mesh = plsc.VectorSubcoreMesh(core_axis_name="c", subcore_axis_name="s")

    @functools.partial(
        pl.kernel,
        out_type=jax.ShapeDtypeStruct((_NW, n), jnp.float32),
        mesh=mesh,
        scratch_types=[
            pltpu.VMEM((nchunk, ch), jnp.int32),
            pltpu.VMEM((n,), jnp.float32),
        ],
        compiler_params=pltpu.CompilerParams(needs_layout_passes=False),
    )
    def deg(dst_hbm, out_hbm, dst_v, deg_v):
        cid = lax.axis_index("c")
        sid = lax.axis_index("s")
        wid = cid * _NS + sid
        pltpu.sync_copy(dst_hbm.at[wid], dst_v)

        zero = jnp.zeros((_LANES,), jnp.float32)

        def zbody(i, _):
            deg_v[pl.ds(i * _LANES, _LANES)] = zero
            return 0

        lax.fori_loop(0, n // _LANES, zbody, 0)

        ones = jnp.ones((_LANES,), jnp.float32)
        groups = ch // _LANES

        def body(i, _):
            r = i // groups
            g = i % groups
            idx = dst_v[r, pl.ds(g * _LANES, _LANES)]
            plsc.addupdate_scatter(deg_v, [idx], ones)
            return 0

        lax.fori_loop(0, nchunk * groups, body, 0)
        pltpu.sync_copy(deg_v, out_hbm.at[wid])

    return deg


def _pick_chunk(per_w):
    for c in range(128, 7, -8):
        if per_w % c == 0:
            return c
    return 0


def kernel(ent_emb, rel_emb, edge_index, cross_edge_index, edge_attr,
           rel_inv_W, rel_inv_b, bn_gamma, bn_beta, rel_lin_W, rel_lin_b,
           ent_lin_W, ent_lin_b, s_lin_W, s_lin_b, conv_Wg, conv_bg,
           conv_Wo, conv_Ws, s_conv_Wg, s_conv_bg, s_conv_Wo, s_conv_Ws):
    n, d = ent_emb.shape
    e = edge_index.shape[1]
    r2 = 2 * rel_emb.shape[0]

    g, rel_final, last_rel = pl.pallas_call(
        _rel_kernel,
        out_shape=[
            jax.ShapeDtypeStruct((4, r2, d), jnp.float32),
            jax.ShapeDtypeStruct((r2, 3 * d), jnp.float32),
            jax.ShapeDtypeStruct((r2, d), jnp.float32),
        ],
    )(rel_emb, rel_inv_W, rel_inv_b, rel_lin_W, rel_lin_b,
      conv_Wg, conv_bg, s_conv_Wg, s_conv_bg)

    per_w = e // _NW
    ch = _pick_chunk(per_w)
    pad = 0
    if ch == 0 or e % _NW:
        per_w = -(-e // _NW)
        per_w += (-per_w) % 8
        ch = _pick_chunk(per_w) or 8
        pad = per_w * _NW - e
    n_acc = -(-(n + (1 if pad else 0)) // 128) * 128
    nchunk = per_w // ch
    cb = _block_div(nchunk)
    nblk = nchunk // cb

    def _stage(a, fill):
        if pad:
            a = jnp.concatenate(
                [a, jnp.full((pad,), fill, jnp.int32)])
        return a.reshape(_NW, nblk, cb, ch)

    src_e = _stage(edge_index[0], 0)
    dst_e = _stage(edge_index[1], n)
    src_c = _stage(cross_edge_index[0], 0)
    dst_c = _stage(cross_edge_index[1], n)
    attr = _stage(edge_attr, 0)

    deg_fn = _make_deg_sc(n_acc, nchunk, ch)
    degp_e = deg_fn(dst_e.reshape(_NW, nchunk, ch))[:, :n]
    degp_c = deg_fn(dst_c.reshape(_NW, nchunk, ch))[:, :n]
    dinv_e, dinv_c = pl.pallas_call(
        _deginv_kernel,
        out_shape=[jax.ShapeDtypeStruct((n, 1), jnp.float32)] * 2,
    )(degp_e, degp_c)

    conv_fn = _make_conv_sc(n_acc, d, nblk, cb, ch)

    nb = 1000 if n % 1000 == 0 else n
    grid = (n // nb,)
    wspec = pl.BlockSpec((d, d), lambda i: (0, 0))
    xspec = pl.BlockSpec((nb, d), lambda i: (i, 0))
    accspec = pl.BlockSpec((_NC, nb, d), lambda i: (0, i, 0))
    dspec = pl.BlockSpec((nb, 1), lambda i: (i, 0))

    comb = pl.pallas_call(
        _comb_kernel,
        grid=grid,
        in_specs=[accspec, accspec, dspec, dspec, xspec, xspec,
                  wspec, wspec, wspec, wspec],
        out_specs=[xspec, xspec, xspec, xspec],
        out_shape=[jax.ShapeDtypeStruct((n, d), jnp.float32)] * 4,
    )

    out = ent_emb
    s_out = ent_emb
    norms = []
    for i in range(bn_gamma.shape[0]):
        lx, slx = pl.pallas_call(
            _bnlin_kernel,
            out_shape=[jax.ShapeDtypeStruct((n, d), jnp.float32)] * 2,
        )(out, s_out, bn_gamma[i], bn_beta[i],
          ent_lin_W[i], ent_lin_b[i], s_lin_W[i], s_lin_b[i])

        acc1 = conv_fn(lx, g[2 * i], src_e, dst_e, attr)[:, :n]
        acc2 = conv_fn(slx, g[2 * i + 1], src_c, dst_c, attr)[:, :n]

        out, s_out, n_out, n_s = comb(
            acc1, acc2, dinv_e, dinv_c, lx, slx,
            conv_Wo[i], conv_Ws[i], s_conv_Wo[i], s_conv_Ws[i])
        norms.append(n_out)
        norms.append(n_s)

    final, last_ent = pl.pallas_call(
        _final_kernel,
        grid=grid,
        in_specs=[xspec] * 5,
        out_specs=[pl.BlockSpec((nb, 5 * d), lambda i: (i, 0)),
                   pl.BlockSpec((nb, 2 * d), lambda i: (i, 0))],
        out_shape=[jax.ShapeDtypeStruct((n, 5 * d), jnp.float32),
                   jax.ShapeDtypeStruct((n, 2 * d), jnp.float32)],
    )(ent_emb, norms[0], norms[1], norms[2], norms[3])

    return final, rel_final, last_ent, last_rel
